# baseline (speedup 1.0000x reference)
import functools

import jax
import jax.numpy as jnp
from jax import lax
from jax.experimental import pallas as pl
from jax.experimental.pallas import tpu as pltpu
from jax.experimental.pallas import tpu_sc as plsc

TAU = 0.5
CAP_FACTOR = 1.25
LANES = 128


def _router_body(cap, n_exp, trash, h_ref, wr_ref, br_ref, p_ref, cnt_ref):
    i = pl.program_id(0)

    @pl.when(i == 0)
    def _init():
        cnt_ref[...] = jnp.zeros_like(cnt_ref)

    h = h_ref[...]
    logits = lax.dot_general(h, wr_ref[...], (((1,), (0,)), ((), ())),
                             preferred_element_type=jnp.float32)
    logits = logits + br_ref[...]
    tb = logits.shape[0]
    lane = lax.broadcasted_iota(jnp.int32, logits.shape, 1)
    m = jnp.max(logits, axis=1, keepdims=True)
    top1 = jnp.min(jnp.where(logits == m, lane, LANES), axis=1)

    d = (lane == top1[:, None]).astype(jnp.float32)
    row = lax.broadcasted_iota(jnp.int32, (tb, tb), 0)
    col = lax.broadcasted_iota(jnp.int32, (tb, tb), 1)
    tri = (col <= row).astype(jnp.float32)
    ranks = lax.dot_general(tri, d, (((1,), (0,)), ((), ())),
                            preferred_element_type=jnp.float32)
    cnt = cnt_ref[...]
    rank_tok = jnp.sum((ranks + cnt) * d, axis=1).astype(jnp.int32)
    cnt_ref[...] = cnt + jnp.sum(d, axis=0, keepdims=True)

    slot = jnp.where(rank_tok <= cap, top1 * cap + rank_tok - 1, trash)
    p_ref[...] = slot.reshape(1, 1, tb)


def _ffn_body(nb_tok, h_ref, w1_ref, w2_ref, th_ref, y_ref):
    i = pl.program_id(0)

    @pl.when(i < nb_tok)
    def _compute():
        xb = h_ref[...]
        z = lax.dot_general(xb, w1_ref[0], (((1,), (0,)), ((), ())),
                            preferred_element_type=jnp.float32)
        s = (z / TAU >= th_ref[0]).astype(jnp.bfloat16)
        y = lax.dot_general(s, w2_ref[0], (((1,), (0,)), ((), ())),
                            preferred_element_type=jnp.float32)
        y_ref[...] = y

    @pl.when(i >= nb_tok)
    def _zeros():
        y_ref[...] = jnp.zeros_like(y_ref)


def _make_router(n_tok, d_model, cap, n_exp, trash, tb):
    grid = (n_tok // tb,)
    return pl.pallas_call(
        functools.partial(_router_body, cap, n_exp, trash),
        grid=grid,
        in_specs=[
            pl.BlockSpec((tb, d_model), lambda i: (i, 0)),
            pl.BlockSpec((d_model, LANES), lambda i: (0, 0)),
            pl.BlockSpec((1, LANES), lambda i: (0, 0)),
        ],
        out_specs=pl.BlockSpec((1, 1, tb), lambda i: (i, 0, 0)),
        out_shape=jax.ShapeDtypeStruct((n_tok // tb, 1, tb), jnp.int32),
        scratch_shapes=[pltpu.VMEM((1, LANES), jnp.float32)],
    )


def _make_ffn(s_pad, d_model, d_ff, n_exp, cap, tbf):
    nb_tok = (s_pad // tbf) - 1
    blocks_per_e = cap // tbf

    def e_of(i):
        return jnp.minimum(i // blocks_per_e, n_exp - 1)

    return pl.pallas_call(
        functools.partial(_ffn_body, nb_tok),
        grid=(s_pad // tbf,),
        in_specs=[
            pl.BlockSpec((tbf, d_model), lambda i: (i, 0)),
            pl.BlockSpec((1, d_model, d_ff), lambda i: (e_of(i), 0, 0)),
            pl.BlockSpec((1, d_ff, d_model), lambda i: (e_of(i), 0, 0)),
            pl.BlockSpec(memory_space=pltpu.SMEM),
        ],
        out_specs=pl.BlockSpec((tbf, d_model), lambda i: (i, 0)),
        out_shape=jax.ShapeDtypeStruct((s_pad, d_model), jnp.float32),
    )


def _make_dispatch(n_tok, d_model, s_pad, chunk):
    info = plsc.get_sparse_core_info()
    nw = info.num_cores * info.num_subcores
    per_w = n_tok // nw
    mesh = plsc.VectorSubcoreMesh(core_axis_name="c", subcore_axis_name="s")

    @functools.partial(
        pl.kernel,
        mesh=mesh,
        out_type=jax.ShapeDtypeStruct((s_pad, d_model), jnp.float32),
        scratch_types=[
            pltpu.VMEM((chunk,), jnp.int32),
            pltpu.VMEM((chunk, d_model), jnp.float32),
            pltpu.SemaphoreType.DMA,
        ],
    )
    def dispatch(h_hbm, p_hbm, xb_hbm, idx_v, rows_v, sem):
        wid = lax.axis_index("s") * info.num_cores + lax.axis_index("c")
        base = wid * per_w
        for c in range(per_w // chunk):
            off = base + c * chunk
            pltpu.sync_copy(p_hbm.at[pl.ds(off, chunk)], idx_v)
            pltpu.sync_copy(h_hbm.at[pl.ds(off, chunk)], rows_v)
            pltpu.async_copy(rows_v, xb_hbm.at[idx_v], sem).wait()

    return dispatch


def _make_combine(n_tok, d_model, s_pad, chunk):
    info = plsc.get_sparse_core_info()
    nw = info.num_cores * info.num_subcores
    per_w = n_tok // nw
    mesh = plsc.VectorSubcoreMesh(core_axis_name="c", subcore_axis_name="s")

    @functools.partial(
        pl.kernel,
        mesh=mesh,
        out_type=jax.ShapeDtypeStruct((n_tok, d_model), jnp.float32),
        scratch_types=[
            pltpu.VMEM((chunk,), jnp.int32),
            pltpu.VMEM((chunk, d_model), jnp.float32),
            pltpu.SemaphoreType.DMA,
        ],
    )
    def combine(yb_hbm, p_hbm, y_hbm, idx_v, rows_v, sem):
        wid = lax.axis_index("s") * info.num_cores + lax.axis_index("c")
        base = wid * per_w
        for c in range(per_w // chunk):
            off = base + c * chunk
            pltpu.sync_copy(p_hbm.at[pl.ds(off, chunk)], idx_v)
            pltpu.async_copy(yb_hbm.at[idx_v], rows_v, sem).wait()
            pltpu.sync_copy(rows_v, y_hbm.at[pl.ds(off, chunk)])

    return combine


def kernel(x, Wr, br, W1, W2, theta):
    B, T, D = x.shape
    N = B * T
    E = Wr.shape[1]
    F = W1.shape[2]
    cap = int(CAP_FACTOR * (N / E) + 1e-06)

    tbf = 256
    s_real = E * cap
    s_pad = s_real + tbf
    trash = s_real

    h = x.reshape(N, D)
    wr_pad = jnp.zeros((D, LANES), Wr.dtype).at[:, :E].set(Wr)
    br_pad = jnp.full((1, LANES), -1e30, jnp.float32).at[0, :E].set(br)

    p = _make_router(N, D, cap, E, trash, 1024)(h, wr_pad, br_pad)
    p = p.reshape(N)
    xb = _make_dispatch(N, D, s_pad, 64)(h, p)
    yb = _make_ffn(s_pad, D, F, E, cap, tbf)(xb, W1, W2.astype(jnp.bfloat16),
                                             theta)
    y = _make_combine(N, D, s_pad, 64)(yb, p)
    return y.reshape(B, T, D)


# speedup vs baseline: 1.0410x; 1.0410x over previous
import functools

import jax
import jax.numpy as jnp
from jax import lax
from jax.experimental import pallas as pl
from jax.experimental.pallas import tpu as pltpu
from jax.experimental.pallas import tpu_sc as plsc

TAU = 0.5
CAP_FACTOR = 1.25
LANES = 128


def _router_body(cap, n_exp, trash, h_ref, wr_ref, br_ref, p_ref, cnt_ref):
    i = pl.program_id(0)

    @pl.when(i == 0)
    def _init():
        cnt_ref[...] = jnp.zeros_like(cnt_ref)

    h = h_ref[...]
    logits = lax.dot_general(h, wr_ref[...], (((1,), (0,)), ((), ())),
                             preferred_element_type=jnp.float32)
    logits = logits + br_ref[...]
    tb = logits.shape[0]
    lane = lax.broadcasted_iota(jnp.int32, logits.shape, 1)
    m = jnp.max(logits, axis=1, keepdims=True)
    top1 = jnp.min(jnp.where(logits == m, lane, LANES), axis=1)

    d = (lane == top1[:, None]).astype(jnp.float32)
    row = lax.broadcasted_iota(jnp.int32, (tb, tb), 0)
    col = lax.broadcasted_iota(jnp.int32, (tb, tb), 1)
    tri = (col <= row).astype(jnp.float32)
    ranks = lax.dot_general(tri, d, (((1,), (0,)), ((), ())),
                            preferred_element_type=jnp.float32)
    cnt = cnt_ref[...]
    rank_tok = jnp.sum((ranks + cnt) * d, axis=1).astype(jnp.int32)
    cnt_ref[...] = cnt + jnp.sum(d, axis=0, keepdims=True)

    slot = jnp.where(rank_tok <= cap, top1 * cap + rank_tok - 1, trash)
    p_ref[...] = slot.reshape(1, 1, tb)


def _ffn_body(nb_tok, h_ref, w1_ref, w2_ref, th_ref, y_ref):
    i = pl.program_id(0)

    @pl.when(i < nb_tok)
    def _compute():
        xb = h_ref[...]
        z = lax.dot_general(xb, w1_ref[0], (((1,), (0,)), ((), ())),
                            preferred_element_type=jnp.float32)
        s = (z / TAU >= th_ref[0]).astype(jnp.float32)
        y = lax.dot_general(s, w2_ref[0], (((1,), (0,)), ((), ())),
                            preferred_element_type=jnp.float32)
        y_ref[...] = y

    @pl.when(i >= nb_tok)
    def _zeros():
        y_ref[...] = jnp.zeros_like(y_ref)


def _make_router(n_tok, d_model, cap, n_exp, trash, tb):
    grid = (n_tok // tb,)
    return pl.pallas_call(
        functools.partial(_router_body, cap, n_exp, trash),
        grid=grid,
        in_specs=[
            pl.BlockSpec((tb, d_model), lambda i: (i, 0)),
            pl.BlockSpec((d_model, LANES), lambda i: (0, 0)),
            pl.BlockSpec((1, LANES), lambda i: (0, 0)),
        ],
        out_specs=pl.BlockSpec((1, 1, tb), lambda i: (i, 0, 0)),
        out_shape=jax.ShapeDtypeStruct((n_tok // tb, 1, tb), jnp.int32),
        scratch_shapes=[pltpu.VMEM((1, LANES), jnp.float32)],
    )


def _make_ffn(s_pad, d_model, d_ff, n_exp, cap, tbf):
    nb_tok = (s_pad // tbf) - 1
    blocks_per_e = cap // tbf

    def e_of(i):
        return jnp.minimum(i // blocks_per_e, n_exp - 1)

    return pl.pallas_call(
        functools.partial(_ffn_body, nb_tok),
        grid=(s_pad // tbf,),
        in_specs=[
            pl.BlockSpec((tbf, d_model), lambda i: (i, 0)),
            pl.BlockSpec((1, d_model, d_ff), lambda i: (e_of(i), 0, 0)),
            pl.BlockSpec((1, d_ff, d_model), lambda i: (e_of(i), 0, 0)),
            pl.BlockSpec(memory_space=pltpu.SMEM),
        ],
        out_specs=pl.BlockSpec((tbf, d_model), lambda i: (i, 0)),
        out_shape=jax.ShapeDtypeStruct((s_pad, d_model), jnp.float32),
    )


def _make_dispatch(n_tok, d_model, s_pad, chunk):
    info = plsc.get_sparse_core_info()
    nw = info.num_cores * info.num_subcores
    per_w = n_tok // nw
    nch = per_w // chunk
    mesh = plsc.VectorSubcoreMesh(core_axis_name="c", subcore_axis_name="s")

    @functools.partial(
        pl.kernel,
        mesh=mesh,
        out_type=jax.ShapeDtypeStruct((s_pad, d_model), jnp.float32),
        scratch_types=[
            pltpu.VMEM((nch, chunk), jnp.int32),
            pltpu.VMEM((2, chunk, d_model), jnp.float32),
            pltpu.SemaphoreType.DMA,
            pltpu.SemaphoreType.DMA,
            pltpu.SemaphoreType.DMA,
            pltpu.SemaphoreType.DMA,
        ],
    )
    def dispatch(h_hbm, p_hbm, xb_hbm, idx_v, rows_v, sl0, sl1, ss0, ss1):
        wid = lax.axis_index("s") * info.num_cores + lax.axis_index("c")
        base = wid * per_w
        pltpu.sync_copy(p_hbm.at[wid], idx_v)
        sem_l, sem_s = [sl0, sl1], [ss0, ss1]

        def load(c):
            off = base + c * chunk
            return pltpu.async_copy(h_hbm.at[pl.ds(off, chunk)],
                                    rows_v.at[c % 2], sem_l[c % 2])

        loads = {0: load(0)}
        scats = [None, None]
        for c in range(nch):
            b = c % 2
            if c + 1 < nch:
                if scats[1 - b] is not None:
                    scats[1 - b].wait()
                loads[c + 1] = load(c + 1)
            loads[c].wait()
            scats[b] = pltpu.async_copy(rows_v.at[b], xb_hbm.at[idx_v.at[c]],
                                        sem_s[b])
        for sc in scats:
            if sc is not None:
                sc.wait()

    return dispatch


def _make_combine(n_tok, d_model, s_pad, chunk):
    info = plsc.get_sparse_core_info()
    nw = info.num_cores * info.num_subcores
    per_w = n_tok // nw
    nch = per_w // chunk
    mesh = plsc.VectorSubcoreMesh(core_axis_name="c", subcore_axis_name="s")

    @functools.partial(
        pl.kernel,
        mesh=mesh,
        out_type=jax.ShapeDtypeStruct((n_tok, d_model), jnp.float32),
        scratch_types=[
            pltpu.VMEM((nch, chunk), jnp.int32),
            pltpu.VMEM((2, chunk, d_model), jnp.float32),
            pltpu.SemaphoreType.DMA,
            pltpu.SemaphoreType.DMA,
            pltpu.SemaphoreType.DMA,
        ],
    )
    def combine(yb_hbm, p_hbm, y_hbm, idx_v, rows_v, sg, ss0, ss1):
        wid = lax.axis_index("s") * info.num_cores + lax.axis_index("c")
        base = wid * per_w
        pltpu.sync_copy(p_hbm.at[wid], idx_v)
        sem_s = [ss0, ss1]
        stores = [None, None]
        for c in range(nch):
            b = c % 2
            if stores[b] is not None:
                stores[b].wait()
            pltpu.async_copy(yb_hbm.at[idx_v.at[c]], rows_v.at[b], sg).wait()
            off = base + c * chunk
            stores[b] = pltpu.async_copy(rows_v.at[b],
                                         y_hbm.at[pl.ds(off, chunk)], sem_s[b])
        for st in stores:
            if st is not None:
                st.wait()

    return combine


def kernel(x, Wr, br, W1, W2, theta):
    B, T, D = x.shape
    N = B * T
    E = Wr.shape[1]
    F = W1.shape[2]
    cap = int(CAP_FACTOR * (N / E) + 1e-06)

    tbf = 256
    s_real = E * cap
    s_pad = s_real + tbf
    trash = s_real

    h = x.reshape(N, D)
    wr_pad = jnp.zeros((D, LANES), Wr.dtype).at[:, :E].set(Wr)
    br_pad = jnp.full((1, LANES), -1e30, jnp.float32).at[0, :E].set(br)

    chunk = 32
    nw = 32
    p = _make_router(N, D, cap, E, trash, 1024)(h, wr_pad, br_pad)
    p = p.reshape(nw, (N // nw) // chunk, chunk)
    xb = _make_dispatch(N, D, s_pad, chunk)(h, p)
    yb = _make_ffn(s_pad, D, F, E, cap, tbf)(xb, W1, W2, theta)
    y = _make_combine(N, D, s_pad, chunk)(yb, p)
    return y.reshape(B, T, D)


# speedup vs baseline: 1.0833x; 1.0406x over previous
import functools

import jax
import jax.numpy as jnp
from jax import lax
from jax.experimental import pallas as pl
from jax.experimental.pallas import tpu as pltpu
from jax.experimental.pallas import tpu_sc as plsc

TAU = 0.5
CAP_FACTOR = 1.25
LANES = 128


def _router_body(cap, n_exp, trash, h_ref, wr_ref, br_ref, p_ref, cnt_ref):
    i = pl.program_id(0)

    @pl.when(i == 0)
    def _init():
        cnt_ref[...] = jnp.zeros_like(cnt_ref)

    h = h_ref[...]
    logits = lax.dot_general(h, wr_ref[...], (((1,), (0,)), ((), ())),
                             preferred_element_type=jnp.float32)
    logits = logits + br_ref[...]
    tb = logits.shape[0]
    lane = lax.broadcasted_iota(jnp.int32, logits.shape, 1)
    m = jnp.max(logits, axis=1, keepdims=True)
    top1 = jnp.min(jnp.where(logits == m, lane, LANES), axis=1)

    d = (lane == top1[:, None]).astype(jnp.float32)
    row = lax.broadcasted_iota(jnp.int32, (tb, tb), 0)
    col = lax.broadcasted_iota(jnp.int32, (tb, tb), 1)
    tri = (col <= row).astype(jnp.float32)
    ranks = lax.dot_general(tri, d, (((1,), (0,)), ((), ())),
                            preferred_element_type=jnp.float32)
    cnt = cnt_ref[...]
    rank_tok = jnp.sum((ranks + cnt) * d, axis=1).astype(jnp.int32)
    cnt_ref[...] = cnt + jnp.sum(d, axis=0, keepdims=True)

    slot = jnp.where(rank_tok <= cap, top1 * cap + rank_tok - 1, trash)
    p_ref[...] = slot.reshape(1, 1, tb)


def _ffn_body(n_exp, h_ref, w1_ref, w2_ref, th_ref, y_ref):
    e = pl.program_id(0)
    fb = pl.program_id(1)

    @pl.when(e < n_exp)
    def _compute():
        xb = h_ref[...]
        z = lax.dot_general(xb, w1_ref[0], (((1,), (0,)), ((), ())),
                            preferred_element_type=jnp.float32)
        s = (z / TAU >= th_ref[e]).astype(jnp.float32)
        part = lax.dot_general(s, w2_ref[0], (((1,), (0,)), ((), ())),
                               preferred_element_type=jnp.float32)

        @pl.when(fb == 0)
        def _set():
            y_ref[...] = part

        @pl.when(fb > 0)
        def _acc():
            y_ref[...] += part

    @pl.when(jnp.logical_and(e == n_exp, fb == 0))
    def _zeros():
        y_ref[...] = jnp.zeros_like(y_ref)


def _make_router(n_tok, d_model, cap, n_exp, trash, tb):
    grid = (n_tok // tb,)
    return pl.pallas_call(
        functools.partial(_router_body, cap, n_exp, trash),
        grid=grid,
        in_specs=[
            pl.BlockSpec((tb, d_model), lambda i: (i, 0)),
            pl.BlockSpec((d_model, LANES), lambda i: (0, 0)),
            pl.BlockSpec((1, LANES), lambda i: (0, 0)),
        ],
        out_specs=pl.BlockSpec((1, 1, tb), lambda i: (i, 0, 0)),
        out_shape=jax.ShapeDtypeStruct((n_tok // tb, 1, tb), jnp.int32),
        scratch_shapes=[pltpu.VMEM((1, LANES), jnp.float32)],
    )


def _make_ffn(s_pad, d_model, d_ff, n_exp, cap, fbs):
    nfb = d_ff // fbs

    def e_w(e, fb):
        return (jnp.minimum(e, n_exp - 1), fb)

    return pl.pallas_call(
        functools.partial(_ffn_body, n_exp),
        grid=(n_exp + 1, nfb),
        in_specs=[
            pl.BlockSpec((cap, d_model), lambda e, fb: (e, 0)),
            pl.BlockSpec((1, d_model, fbs),
                         lambda e, fb: (e_w(e, fb)[0], 0, fb)),
            pl.BlockSpec((1, fbs, d_model),
                         lambda e, fb: (e_w(e, fb)[0], fb, 0)),
            pl.BlockSpec(memory_space=pltpu.SMEM),
        ],
        out_specs=pl.BlockSpec((cap, d_model), lambda e, fb: (e, 0)),
        out_shape=jax.ShapeDtypeStruct((s_pad, d_model), jnp.float32),
    )


def _make_dispatch(n_tok, d_model, s_pad, chunk):
    info = plsc.get_sparse_core_info()
    nw = info.num_cores * info.num_subcores
    per_w = n_tok // nw
    nch = per_w // chunk
    mesh = plsc.VectorSubcoreMesh(core_axis_name="c", subcore_axis_name="s")

    @functools.partial(
        pl.kernel,
        mesh=mesh,
        out_type=jax.ShapeDtypeStruct((s_pad, d_model), jnp.float32),
        scratch_types=[
            pltpu.VMEM((nch, chunk), jnp.int32),
            pltpu.VMEM((2, chunk, d_model), jnp.float32),
            pltpu.SemaphoreType.DMA,
            pltpu.SemaphoreType.DMA,
            pltpu.SemaphoreType.DMA,
            pltpu.SemaphoreType.DMA,
        ],
    )
    def dispatch(h_hbm, p_hbm, xb_hbm, idx_v, rows_v, sl0, sl1, ss0, ss1):
        wid = lax.axis_index("s") * info.num_cores + lax.axis_index("c")
        base = wid * per_w
        pltpu.sync_copy(p_hbm.at[wid], idx_v)
        sem_l, sem_s = [sl0, sl1], [ss0, ss1]

        def load(c):
            off = base + c * chunk
            return pltpu.async_copy(h_hbm.at[pl.ds(off, chunk)],
                                    rows_v.at[c % 2], sem_l[c % 2])

        loads = {0: load(0)}
        scats = [None, None]
        for c in range(nch):
            b = c % 2
            if c + 1 < nch:
                if scats[1 - b] is not None:
                    scats[1 - b].wait()
                loads[c + 1] = load(c + 1)
            loads[c].wait()
            scats[b] = pltpu.async_copy(rows_v.at[b], xb_hbm.at[idx_v.at[c]],
                                        sem_s[b])
        for sc in scats:
            if sc is not None:
                sc.wait()

    return dispatch


def _make_combine(n_tok, d_model, s_pad, chunk):
    info = plsc.get_sparse_core_info()
    nw = info.num_cores * info.num_subcores
    per_w = n_tok // nw
    nch = per_w // chunk
    mesh = plsc.VectorSubcoreMesh(core_axis_name="c", subcore_axis_name="s")

    @functools.partial(
        pl.kernel,
        mesh=mesh,
        out_type=jax.ShapeDtypeStruct((n_tok, d_model), jnp.float32),
        scratch_types=[
            pltpu.VMEM((nch, chunk), jnp.int32),
            pltpu.VMEM((2, chunk, d_model), jnp.float32),
            pltpu.SemaphoreType.DMA,
            pltpu.SemaphoreType.DMA,
            pltpu.SemaphoreType.DMA,
        ],
    )
    def combine(yb_hbm, p_hbm, y_hbm, idx_v, rows_v, sg, ss0, ss1):
        wid = lax.axis_index("s") * info.num_cores + lax.axis_index("c")
        base = wid * per_w
        pltpu.sync_copy(p_hbm.at[wid], idx_v)
        sem_s = [ss0, ss1]
        stores = [None, None]
        for c in range(nch):
            b = c % 2
            if stores[b] is not None:
                stores[b].wait()
            pltpu.async_copy(yb_hbm.at[idx_v.at[c]], rows_v.at[b], sg).wait()
            off = base + c * chunk
            stores[b] = pltpu.async_copy(rows_v.at[b],
                                         y_hbm.at[pl.ds(off, chunk)], sem_s[b])
        for st in stores:
            if st is not None:
                st.wait()

    return combine


def kernel(x, Wr, br, W1, W2, theta):
    B, T, D = x.shape
    N = B * T
    E = Wr.shape[1]
    F = W1.shape[2]
    cap = int(CAP_FACTOR * (N / E) + 1e-06)

    fbs = 512
    s_real = E * cap
    s_pad = s_real + cap
    trash = s_real

    h = x.reshape(N, D)
    wr_pad = jnp.zeros((D, LANES), Wr.dtype).at[:, :E].set(Wr)
    br_pad = jnp.full((1, LANES), -1e30, jnp.float32).at[0, :E].set(br)

    chunk = 32
    nw = 32
    p = _make_router(N, D, cap, E, trash, 1024)(h, wr_pad, br_pad)
    p = p.reshape(nw, (N // nw) // chunk, chunk)
    xb = _make_dispatch(N, D, s_pad, chunk)(h, p)
    yb = _make_ffn(s_pad, D, F, E, cap, fbs)(xb, W1, W2, theta)
    y = _make_combine(N, D, s_pad, chunk)(yb, p)
    return y.reshape(B, T, D)
